# baseline (speedup 1.0000x reference)
import jax
import jax.numpy as jnp
from jax.experimental import pallas as pl
from jax.experimental.pallas import tpu as pltpu

ISIZE = 224
PSIZE = 30
ROW0 = 172
COL0 = 47
NBULK_TOP = 8
NBULK_BOT = 4
NBAND = 8
ROWA = 168
BANDROWS = 40


def _patch_kernel(x_hbm, patch_hbm, out_hbm, band_vmem, patch_vmem,
                  canvas_vmem, sem_bulk, sem_in, sem_out, sem_p):
    batch = x_hbm.shape[0]

    cp_p = pltpu.make_async_copy(patch_hbm, patch_vmem, sem_p)
    cp_p.start()

    band_in = []
    for k in range(NBAND):
        bb = batch // NBAND
        c = pltpu.make_async_copy(
            x_hbm.at[pl.ds(k * bb, bb), :, pl.ds(ROWA, BANDROWS), :],
            band_vmem.at[pl.ds(k * bb, bb)],
            sem_in)
        c.start()
        band_in.append(c)

    bulk = []
    for k in range(NBULK_TOP):
        bb = batch // NBULK_TOP
        c = pltpu.make_async_copy(
            x_hbm.at[pl.ds(k * bb, bb), :, pl.ds(0, ROWA), :],
            out_hbm.at[pl.ds(k * bb, bb), :, pl.ds(0, ROWA), :],
            sem_bulk)
        c.start()
        bulk.append(c)
    for k in range(NBULK_BOT):
        bb = batch // NBULK_BOT
        c = pltpu.make_async_copy(
            x_hbm.at[pl.ds(k * bb, bb), :, pl.ds(ROWA + BANDROWS, ISIZE - ROWA - BANDROWS), :],
            out_hbm.at[pl.ds(k * bb, bb), :, pl.ds(ROWA + BANDROWS, ISIZE - ROWA - BANDROWS), :],
            sem_bulk)
        c.start()
        bulk.append(c)

    cp_p.wait()
    canvas_vmem[...] = jnp.zeros_like(canvas_vmem)
    canvas_vmem[:, :, ROW0 - ROWA:ROW0 - ROWA + PSIZE, COL0:COL0 + PSIZE] = (
        patch_vmem[...])

    for c in band_in:
        c.wait()
    band_vmem[:, :, :, 0:128] = band_vmem[:, :, :, 0:128] + canvas_vmem[:, :, :, 0:128]

    band_out = []
    for k in range(NBAND):
        bb = batch // NBAND
        c = pltpu.make_async_copy(
            band_vmem.at[pl.ds(k * bb, bb)],
            out_hbm.at[pl.ds(k * bb, bb), :, pl.ds(ROWA, BANDROWS), :],
            sem_out)
        c.start()
        band_out.append(c)

    for c in bulk:
        c.wait()
    for c in band_out:
        c.wait()


def kernel(x, patch):
    batch = x.shape[0]
    return pl.pallas_call(
        _patch_kernel,
        in_specs=[
            pl.BlockSpec(memory_space=pl.MemorySpace.ANY),
            pl.BlockSpec(memory_space=pl.MemorySpace.ANY),
        ],
        out_specs=pl.BlockSpec(memory_space=pl.MemorySpace.ANY),
        out_shape=jax.ShapeDtypeStruct(x.shape, x.dtype),
        scratch_shapes=[
            pltpu.VMEM((batch, 3, BANDROWS, ISIZE), jnp.float32),
            pltpu.VMEM((1, 3, PSIZE, PSIZE), jnp.float32),
            pltpu.VMEM((1, 3, BANDROWS, ISIZE), jnp.float32),
            pltpu.SemaphoreType.DMA,
            pltpu.SemaphoreType.DMA,
            pltpu.SemaphoreType.DMA,
            pltpu.SemaphoreType.DMA,
        ],
    )(x, patch)


# speedup vs baseline: 10.1579x; 10.1579x over previous
import jax
import jax.numpy as jnp
from jax.experimental import pallas as pl
from jax.experimental.pallas import tpu as pltpu

ISIZE = 224
PSIZE = 30
ROW0 = 172
COL0 = 47
BB = 2


def _add_patch_kernel(x_ref, patch_ref, out_ref):
    out_ref[...] = x_ref[...]
    out_ref[:, :, ROW0:ROW0 + PSIZE, COL0:COL0 + PSIZE] = (
        out_ref[:, :, ROW0:ROW0 + PSIZE, COL0:COL0 + PSIZE] + patch_ref[...]
    )


def kernel(x, patch):
    batch = x.shape[0]
    grid = (batch // BB,)
    return pl.pallas_call(
        _add_patch_kernel,
        grid=grid,
        in_specs=[
            pl.BlockSpec((BB, 3, ISIZE, ISIZE), lambda i: (i, 0, 0, 0)),
            pl.BlockSpec((1, 3, PSIZE, PSIZE), lambda i: (0, 0, 0, 0)),
        ],
        out_specs=pl.BlockSpec((BB, 3, ISIZE, ISIZE), lambda i: (i, 0, 0, 0)),
        out_shape=jax.ShapeDtypeStruct(x.shape, x.dtype),
        compiler_params=pltpu.CompilerParams(
            dimension_semantics=("arbitrary",),
        ),
    )(x, patch)


# speedup vs baseline: 10.8922x; 1.0723x over previous
import jax
import jax.numpy as jnp
from jax.experimental import pallas as pl
from jax.experimental.pallas import tpu as pltpu

ISIZE = 224
PSIZE = 30
ROW0 = 172
COL0 = 47
ROWA = 168
BANDROWS = 40

CB = 2
W = 8
WB = CB * W


def _patch_kernel(x_hbm, patch_hbm, out_hbm, buf_a, buf_b, patch_vmem,
                  canvas_vmem, sem_in_a, sem_in_b, sem_out_a, sem_out_b,
                  sem_p):
    batch = x_hbm.shape[0]
    nwave = batch // WB
    bufs = [buf_a, buf_b]
    sin = [sem_in_a, sem_in_b]
    sout = [sem_out_a, sem_out_b]

    def ins(w):
        buf, sem = bufs[w % 2], sin[w % 2]
        return [pltpu.make_async_copy(
                    x_hbm.at[pl.ds(w * WB + k * CB, CB)],
                    buf.at[pl.ds(k * CB, CB)], sem)
                for k in range(W)]

    def outs(w):
        buf, sem = bufs[w % 2], sout[w % 2]
        return [pltpu.make_async_copy(
                    buf.at[pl.ds(k * CB, CB)],
                    out_hbm.at[pl.ds(w * WB + k * CB, CB)], sem)
                for k in range(W)]

    pltpu.make_async_copy(patch_hbm, patch_vmem, sem_p).start()

    for c in ins(0):
        c.start()

    pltpu.make_async_copy(patch_hbm, patch_vmem, sem_p).wait()
    canvas_vmem[...] = jnp.zeros_like(canvas_vmem)
    canvas_vmem[:, :, ROW0 - ROWA:ROW0 - ROWA + PSIZE, COL0:COL0 + PSIZE] = (
        patch_vmem[...])
    canvas = canvas_vmem[:, :, :, 0:128]

    for w in range(nwave):
        if w >= 1:
            for c in outs(w - 1):
                c.wait()
        if w + 1 < nwave:
            for c in ins(w + 1):
                c.start()
        for c in ins(w):
            c.wait()
        buf = bufs[w % 2]
        buf[:, :, ROWA:ROWA + BANDROWS, 0:128] = (
            buf[:, :, ROWA:ROWA + BANDROWS, 0:128] + canvas)
        for c in outs(w):
            c.start()

    for c in outs(nwave - 1):
        c.wait()


def kernel(x, patch):
    batch = x.shape[0]
    return pl.pallas_call(
        _patch_kernel,
        in_specs=[
            pl.BlockSpec(memory_space=pl.MemorySpace.ANY),
            pl.BlockSpec(memory_space=pl.MemorySpace.ANY),
        ],
        out_specs=pl.BlockSpec(memory_space=pl.MemorySpace.ANY),
        out_shape=jax.ShapeDtypeStruct(x.shape, x.dtype),
        scratch_shapes=[
            pltpu.VMEM((WB, 3, ISIZE, ISIZE), jnp.float32),
            pltpu.VMEM((WB, 3, ISIZE, ISIZE), jnp.float32),
            pltpu.VMEM((1, 3, PSIZE, PSIZE), jnp.float32),
            pltpu.VMEM((1, 3, BANDROWS, ISIZE), jnp.float32),
            pltpu.SemaphoreType.DMA,
            pltpu.SemaphoreType.DMA,
            pltpu.SemaphoreType.DMA,
            pltpu.SemaphoreType.DMA,
            pltpu.SemaphoreType.DMA,
        ],
    )(x, patch)


# speedup vs baseline: 13.3719x; 1.2277x over previous
import jax
import jax.numpy as jnp
from jax.experimental import pallas as pl
from jax.experimental.pallas import tpu as pltpu

ISIZE = 224
PSIZE = 30
ROW0 = 172
COL0 = 47
RB = 56
RBI = 3
R0 = ROW0 - RBI * RB
BB = 16


def _band_kernel(x_ref, patch_ref, out_ref):
    blk = x_ref[...]
    out_ref[...] = blk
    out_ref[:, :, R0:R0 + PSIZE, COL0:COL0 + PSIZE] = (
        blk[:, :, R0:R0 + PSIZE, COL0:COL0 + PSIZE] + patch_ref[...]
    )


def kernel(x, patch):
    batch = x.shape[0]
    grid = (batch // BB,)
    return pl.pallas_call(
        _band_kernel,
        grid=grid,
        in_specs=[
            pl.BlockSpec((BB, 3, RB, ISIZE), lambda i: (i, 0, RBI, 0)),
            pl.BlockSpec((1, 3, PSIZE, PSIZE), lambda i: (0, 0, 0, 0)),
        ],
        out_specs=pl.BlockSpec((BB, 3, RB, ISIZE), lambda i: (i, 0, RBI, 0)),
        out_shape=jax.ShapeDtypeStruct(x.shape, x.dtype),
        input_output_aliases={0: 0},
    )(x, patch)
